# baseline (speedup 1.0000x reference)
import functools

import jax
import jax.numpy as jnp
from jax import lax
from jax.experimental import pallas as pl
from jax.experimental.pallas import tpu as pltpu
from jax.experimental.pallas import tpu_sc as plsc

N = 10000
E = 160000
D = 128
SPLITS = (40960, 40960, 40960, 37120)
SPLIT_OFFS = (0, 40960, 81920, 122880)

NC = 2
NS = 16
NW = NC * NS
CH = 128
ROWS_PER_SUB = 632
NPAD = NS * ROWS_PER_SUB

_f32 = jnp.float32


def _schedule(ne):
    nch = ne // CH
    common = nch // NW
    rem = nch % NW
    kp = common - (common % 2)
    return nch, common, kp, rem


@functools.cache
def _sc_mesh():
    return plsc.VectorSubcoreMesh(core_axis_name="c", subcore_axis_name="s",
                                  num_cores=NC, num_subcores=NS)


@functools.cache
def _gather_kernel(ne, ioff):
    _, common, kp, rem = _schedule(ne)

    def body(p_hbm, q_hbm, src_hbm, dst_hbm, u_hbm,
             is0, is1, id0, id1, rows0, rows1, sg0, sg1, sw0, sw1):
        cid = lax.axis_index("c")
        sid = lax.axis_index("s")
        wid = sid * NC + cid
        idx_s = (is0, is1)
        idx_d = (id0, id1)
        rows = (rows0, rows1)
        sem_g = (sg0, sg1)
        sem_w = (sw0, sw1)

        def obase(k):
            return (wid + k * NW) * CH

        def wait_g(b):
            pltpu.make_async_copy(p_hbm.at[idx_s[b]], rows[b],
                                  sem_g[b]).wait()

        def wait_w(b):
            pltpu.make_async_copy(rows[b], u_hbm.at[pl.ds(0, CH)],
                                  sem_w[b]).wait()

        def prefetch(k, b):
            base = ioff + obase(k)
            pltpu.sync_copy(src_hbm.at[pl.ds(base, CH)], idx_s[b])
            pltpu.sync_copy(dst_hbm.at[pl.ds(base, CH)], idx_d[b])
            pltpu.async_copy(p_hbm.at[idx_s[b]], rows[b], sem_g[b])

        def add_q(b):
            pltpu.async_copy(q_hbm.at[idx_d[b]], rows[b], sem_g[b], add=True)

        def plain_chunk(k, b):
            prefetch(k, b)
            wait_g(b)
            add_q(b)
            wait_g(b)
            pltpu.sync_copy(rows[b], u_hbm.at[pl.ds(obase(k), CH)])

        prefetch(0, 0)

        def outer(t, carry):
            wait_g(0)
            add_q(0)

            @pl.when(t > 0)
            def _():
                wait_w(1)
            prefetch(2 * t + 1, 1)

            wait_g(0)
            pltpu.async_copy(rows[0], u_hbm.at[pl.ds(obase(2 * t), CH)],
                             sem_w[0])

            wait_g(1)
            add_q(1)

            @pl.when(t < kp // 2 - 1)
            def _():
                wait_w(0)
                prefetch(2 * t + 2, 0)

            wait_g(1)
            pltpu.async_copy(rows[1], u_hbm.at[pl.ds(obase(2 * t + 1), CH)],
                             sem_w[1])
            return carry

        lax.fori_loop(0, kp // 2, outer, 0)

        wait_w(0)
        if kp < common:
            plain_chunk(kp, 0)
        wait_w(1)
        if rem > 0:
            @pl.when(wid < rem)
            def _():
                plain_chunk(common, 1)

    return pl.kernel(
        body,
        out_type=jax.ShapeDtypeStruct((ne, D), _f32),
        mesh=_sc_mesh(),
        scratch_types=[
            pltpu.VMEM((CH,), jnp.int32),
            pltpu.VMEM((CH,), jnp.int32),
            pltpu.VMEM((CH,), jnp.int32),
            pltpu.VMEM((CH,), jnp.int32),
            pltpu.VMEM((CH, D), _f32),
            pltpu.VMEM((CH, D), _f32),
            pltpu.SemaphoreType.DMA,
            pltpu.SemaphoreType.DMA,
            pltpu.SemaphoreType.DMA,
            pltpu.SemaphoreType.DMA,
        ],
    )


def _sc_gather(p, q, src, dst, ne, ioff):
    return _gather_kernel(ne, ioff)(p, q, src, dst)


@functools.cache
def _scatter_kernel(ne, ioff, init_per_core):
    _, common, kp, rem = _schedule(ne)

    def body(e_hbm, dst_hbm, init_hbm, out_hbm,
             id0, id1, rows0, rows1, shared, sl0, sl1):
        cid = lax.axis_index("c")
        sid = lax.axis_index("s")
        wid = sid * NC + cid
        idx = (id0, id1)
        rows = (rows0, rows1)
        sem_l = (sl0, sl1)

        def prefetch(k, b):
            obase = (wid + k * NW) * CH
            pltpu.async_copy(dst_hbm.at[pl.ds(ioff + obase, CH)], idx[b],
                             sem_l[b])
            pltpu.async_copy(e_hbm.at[pl.ds(obase, CH)], rows[b], sem_l[b])

        def wait_l(b):
            pltpu.make_async_copy(dst_hbm.at[pl.ds(0, CH)], idx[b],
                                  sem_l[b]).wait()
            pltpu.make_async_copy(e_hbm.at[pl.ds(0, CH)], rows[b],
                                  sem_l[b]).wait()

        def scat(b):
            pltpu.sync_copy(rows[b], shared.at[idx[b]], add=True)

        prefetch(0, 0)
        stripe = pl.ds(sid * ROWS_PER_SUB, ROWS_PER_SUB)
        if init_per_core:
            pltpu.sync_copy(init_hbm.at[cid, stripe], shared.at[stripe])
        else:
            pltpu.sync_copy(init_hbm.at[stripe], shared.at[stripe])
        plsc.subcore_barrier()

        def outer(t, carry):
            wait_l(0)
            prefetch(2 * t + 1, 1)
            scat(0)

            wait_l(1)

            @pl.when(t < kp // 2 - 1)
            def _():
                prefetch(2 * t + 2, 0)
            scat(1)
            return carry

        lax.fori_loop(0, kp // 2, outer, 0)

        if kp < common:
            prefetch(kp, 0)
            wait_l(0)
            if rem > 0:
                @pl.when(wid < rem)
                def _():
                    prefetch(common, 1)
            scat(0)
            if rem > 0:
                @pl.when(wid < rem)
                def _():
                    wait_l(1)
                    scat(1)
        elif rem > 0:
            @pl.when(wid < rem)
            def _():
                prefetch(common, 1)
                wait_l(1)
                scat(1)
        plsc.subcore_barrier()

        pltpu.sync_copy(shared.at[stripe], out_hbm.at[cid, stripe])

    return pl.kernel(
        body,
        out_type=jax.ShapeDtypeStruct((NC, NPAD, D), _f32),
        mesh=_sc_mesh(),
        scratch_types=[
            pltpu.VMEM((CH,), jnp.int32),
            pltpu.VMEM((CH,), jnp.int32),
            pltpu.VMEM((CH, D), _f32),
            pltpu.VMEM((CH, D), _f32),
            pltpu.VMEM_SHARED((NPAD, D), _f32),
            pltpu.SemaphoreType.DMA,
            pltpu.SemaphoreType.DMA,
        ],
    )


def _sc_scatter(e_half, dst, init, ne, ioff, init_per_core):
    return _scatter_kernel(ne, ioff, init_per_core)(e_half, dst, init)


def _dot(a, b):
    return jnp.dot(a, b, preferred_element_type=_f32)


def _proj_body(x_ref, w_ref, p_ref, q_ref):
    pq = _dot(x_ref[...], w_ref[...])
    p_ref[...] = pq[:, :D]
    q_ref[...] = pq[:, D:]


def _proj(x, w_cat, bn=2000):
    grid = (N // bn,)
    return pl.pallas_call(
        _proj_body,
        grid=grid,
        in_specs=[
            pl.BlockSpec((bn, D), lambda i: (i, 0)),
            pl.BlockSpec((D, 2 * D), lambda i: (0, 0)),
        ],
        out_specs=[
            pl.BlockSpec((bn, D), lambda i: (i, 0)),
            pl.BlockSpec((bn, D), lambda i: (i, 0)),
        ],
        out_shape=[
            jax.ShapeDtypeStruct((N, D), _f32),
            jax.ShapeDtypeStruct((N, D), _f32),
        ],
        compiler_params=pltpu.CompilerParams(
            dimension_semantics=("parallel",)),
    )(x, w_cat)


def _mlp_tail(h, w2_ref, b2_ref, w3_ref, b3_ref, g_ref, bt_ref):
    h = jnp.maximum(_dot(h, w2_ref[...]) + b2_ref[...], 0.0)
    h = _dot(h, w3_ref[...]) + b3_ref[...]
    mu = jnp.mean(h, axis=-1, keepdims=True)
    d = h - mu
    var = jnp.mean(d * d, axis=-1, keepdims=True)
    return (d * lax.rsqrt(var + 1e-5)) * g_ref[...] + bt_ref[...]


def _edge_body(e_ref, u_ref, w1_ref, b1_ref, w2_ref, b2_ref,
               w3_ref, b3_ref, g_ref, bt_ref, out_ref):
    h = _dot(e_ref[...], w1_ref[...]) + u_ref[...] + b1_ref[...]
    h = jnp.maximum(h, 0.0)
    out_ref[...] = _mlp_tail(h, w2_ref, b2_ref, w3_ref, b3_ref,
                             g_ref, bt_ref) + e_ref[...]


def _edge(e, u, w1a, b1, w2, b2, w3, b3, g, bt, ne, eoff, be=1280):
    grid = (ne // be,)
    blk = eoff // be
    erow = lambda i: (i + blk, 0)
    row = lambda i: (i, 0)
    full = lambda i: (0, 0)
    return pl.pallas_call(
        _edge_body,
        grid=grid,
        in_specs=[
            pl.BlockSpec((be, D), erow),
            pl.BlockSpec((be, D), row),
            pl.BlockSpec((D, D), full),
            pl.BlockSpec((1, D), full),
            pl.BlockSpec((D, D), full),
            pl.BlockSpec((1, D), full),
            pl.BlockSpec((D, D), full),
            pl.BlockSpec((1, D), full),
            pl.BlockSpec((1, D), full),
            pl.BlockSpec((1, D), full),
        ],
        out_specs=pl.BlockSpec((be, D), row),
        out_shape=jax.ShapeDtypeStruct((ne, D), _f32),
        compiler_params=pltpu.CompilerParams(
            dimension_semantics=("parallel",)),
    )(e, u, w1a, b1, w2, b2, w3, b3, g, bt)


def _node_body(a0_ref, a1_ref, x_ref, wa_ref, wx_ref, b1_ref, w2_ref, b2_ref,
               w3_ref, b3_ref, g_ref, bt_ref, out_ref):
    s = a0_ref[...] + a1_ref[...]
    h = _dot(s, wa_ref[...]) + _dot(x_ref[...], wx_ref[...]) + b1_ref[...]
    h = jnp.maximum(h, 0.0)
    out_ref[...] = _mlp_tail(h, w2_ref, b2_ref, w3_ref, b3_ref,
                             g_ref, bt_ref) + x_ref[...]


def _node(a0, a1, x, wa, wx, b1, w2, b2, w3, b3, g, bt, bn=2000):
    grid = (N // bn,)
    row = lambda i: (i, 0)
    full = lambda i: (0, 0)
    return pl.pallas_call(
        _node_body,
        grid=grid,
        in_specs=[
            pl.BlockSpec((bn, D), row),
            pl.BlockSpec((bn, D), row),
            pl.BlockSpec((bn, D), row),
            pl.BlockSpec((D, D), full),
            pl.BlockSpec((D, D), full),
            pl.BlockSpec((1, D), full),
            pl.BlockSpec((D, D), full),
            pl.BlockSpec((1, D), full),
            pl.BlockSpec((D, D), full),
            pl.BlockSpec((1, D), full),
            pl.BlockSpec((1, D), full),
            pl.BlockSpec((1, D), full),
        ],
        out_specs=pl.BlockSpec((bn, D), row),
        out_shape=jax.ShapeDtypeStruct((N, D), _f32),
        compiler_params=pltpu.CompilerParams(
            dimension_semantics=("parallel",)),
    )(a0, a1, x, wa, wx, b1, w2, b2, w3, b3, g, bt)


def kernel(node_features, edge_features, edge_index,
           e_w1, e_b1, e_w2, e_b2, e_w3, e_b3, e_g, e_bt,
           n_w1, n_b1, n_w2, n_b2, n_w3, n_b3, n_g, n_bt):
    src = edge_index[0]
    dst = edge_index[1]
    x = node_features
    zeros = jnp.zeros((NPAD, D), _f32)
    num_passes = e_w1.shape[0]
    r = lambda v: v.reshape(1, D)

    esplit = [None] * len(SPLITS)
    first = True
    for i in range(num_passes):
        w1 = e_w1[i]
        w1a = w1[:D]
        p, q = _proj(x, jnp.concatenate([w1[D:2 * D], w1[2 * D:]], axis=1))
        ew = (w1a, r(e_b1[i]), e_w2[i], r(e_b2[i]),
              e_w3[i], r(e_b3[i]), r(e_g[i]), r(e_bt[i]))
        us = [_sc_gather(p, q, src, dst, ne, off)
              for ne, off in zip(SPLITS, SPLIT_OFFS)]
        for j, (ne, off) in enumerate(zip(SPLITS, SPLIT_OFFS)):
            if first:
                esplit[j] = _edge(edge_features, us[j], *ew, ne=ne, eoff=off)
            else:
                esplit[j] = _edge(esplit[j], us[j], *ew, ne=ne, eoff=0)
        first = False
        part = zeros
        for j, (ne, off) in enumerate(zip(SPLITS, SPLIT_OFFS)):
            part = _sc_scatter(esplit[j], dst, part, ne, off, j > 0)
        nw1 = n_w1[i]
        x = _node(part[0], part[1], x, nw1[:D], nw1[D:], r(n_b1[i]),
                  n_w2[i], r(n_b2[i]), n_w3[i], r(n_b3[i]),
                  r(n_g[i]), r(n_bt[i]))
    return x


# speedup vs baseline: 1.1140x; 1.1140x over previous
import functools

import jax
import jax.numpy as jnp
from jax import lax
from jax.experimental import pallas as pl
from jax.experimental.pallas import tpu as pltpu
from jax.experimental.pallas import tpu_sc as plsc

N = 10000
E = 160000
D = 128
SPLITS = (80000, 80000)
SPLIT_OFFS = (0, 80000)

NC = 2
NS = 16
NW = NC * NS
CH = 128
ROWS_PER_SUB = 632
NPAD = NS * ROWS_PER_SUB

_f32 = jnp.float32


def _schedule(ne):
    nch = ne // CH
    common = nch // NW
    rem = nch % NW
    kp = common - (common % 2)
    return nch, common, kp, rem


@functools.cache
def _sc_mesh():
    return plsc.VectorSubcoreMesh(core_axis_name="c", subcore_axis_name="s",
                                  num_cores=NC, num_subcores=NS)


@functools.cache
def _gather_kernel(ne, ioff):
    _, common, kp, rem = _schedule(ne)

    def body(p_hbm, q_hbm, src_hbm, dst_hbm, u_hbm,
             is0, is1, id0, id1, rows0, rows1, sg0, sg1, sw0, sw1):
        cid = lax.axis_index("c")
        sid = lax.axis_index("s")
        wid = sid * NC + cid
        idx_s = (is0, is1)
        idx_d = (id0, id1)
        rows = (rows0, rows1)
        sem_g = (sg0, sg1)
        sem_w = (sw0, sw1)

        def obase(k):
            return (wid + k * NW) * CH

        def wait_g(b):
            pltpu.make_async_copy(p_hbm.at[idx_s[b]], rows[b],
                                  sem_g[b]).wait()

        def wait_w(b):
            pltpu.make_async_copy(rows[b], u_hbm.at[pl.ds(0, CH)],
                                  sem_w[b]).wait()

        def prefetch(k, b):
            base = ioff + obase(k)
            pltpu.sync_copy(src_hbm.at[pl.ds(base, CH)], idx_s[b])
            pltpu.sync_copy(dst_hbm.at[pl.ds(base, CH)], idx_d[b])
            pltpu.async_copy(p_hbm.at[idx_s[b]], rows[b], sem_g[b])

        def add_q(b):
            pltpu.async_copy(q_hbm.at[idx_d[b]], rows[b], sem_g[b], add=True)

        def plain_chunk(k, b):
            prefetch(k, b)
            wait_g(b)
            add_q(b)
            wait_g(b)
            pltpu.sync_copy(rows[b], u_hbm.at[pl.ds(obase(k), CH)])

        prefetch(0, 0)

        def outer(t, carry):
            wait_g(0)
            add_q(0)

            @pl.when(t > 0)
            def _():
                wait_w(1)
            prefetch(2 * t + 1, 1)

            wait_g(0)
            pltpu.async_copy(rows[0], u_hbm.at[pl.ds(obase(2 * t), CH)],
                             sem_w[0])

            wait_g(1)
            add_q(1)

            @pl.when(t < kp // 2 - 1)
            def _():
                wait_w(0)
                prefetch(2 * t + 2, 0)

            wait_g(1)
            pltpu.async_copy(rows[1], u_hbm.at[pl.ds(obase(2 * t + 1), CH)],
                             sem_w[1])
            return carry

        lax.fori_loop(0, kp // 2, outer, 0)

        wait_w(0)
        if kp < common:
            plain_chunk(kp, 0)
        wait_w(1)
        if rem > 0:
            @pl.when(wid < rem)
            def _():
                plain_chunk(common, 1)

    return pl.kernel(
        body,
        out_type=jax.ShapeDtypeStruct((ne, D), _f32),
        mesh=_sc_mesh(),
        scratch_types=[
            pltpu.VMEM((CH,), jnp.int32),
            pltpu.VMEM((CH,), jnp.int32),
            pltpu.VMEM((CH,), jnp.int32),
            pltpu.VMEM((CH,), jnp.int32),
            pltpu.VMEM((CH, D), _f32),
            pltpu.VMEM((CH, D), _f32),
            pltpu.SemaphoreType.DMA,
            pltpu.SemaphoreType.DMA,
            pltpu.SemaphoreType.DMA,
            pltpu.SemaphoreType.DMA,
        ],
    )


def _sc_gather(p, q, src, dst, ne, ioff):
    return _gather_kernel(ne, ioff)(p, q, src, dst)


@functools.cache
def _scatter_kernel(ne, ioff, init_per_core):
    _, common, kp, rem = _schedule(ne)

    def body(e_hbm, dst_hbm, init_hbm, out_hbm,
             id0, id1, rows0, rows1, shared, sl0, sl1):
        cid = lax.axis_index("c")
        sid = lax.axis_index("s")
        wid = sid * NC + cid
        idx = (id0, id1)
        rows = (rows0, rows1)
        sem_l = (sl0, sl1)

        def prefetch(k, b):
            obase = (wid + k * NW) * CH
            pltpu.async_copy(dst_hbm.at[pl.ds(ioff + obase, CH)], idx[b],
                             sem_l[b])
            pltpu.async_copy(e_hbm.at[pl.ds(obase, CH)], rows[b], sem_l[b])

        def wait_l(b):
            pltpu.make_async_copy(dst_hbm.at[pl.ds(0, CH)], idx[b],
                                  sem_l[b]).wait()
            pltpu.make_async_copy(e_hbm.at[pl.ds(0, CH)], rows[b],
                                  sem_l[b]).wait()

        def scat(b):
            pltpu.sync_copy(rows[b], shared.at[idx[b]], add=True)

        prefetch(0, 0)
        stripe = pl.ds(sid * ROWS_PER_SUB, ROWS_PER_SUB)
        if init_per_core:
            pltpu.sync_copy(init_hbm.at[cid, stripe], shared.at[stripe])
        else:
            pltpu.sync_copy(init_hbm.at[stripe], shared.at[stripe])
        plsc.subcore_barrier()

        def outer(t, carry):
            wait_l(0)
            prefetch(2 * t + 1, 1)
            scat(0)

            wait_l(1)

            @pl.when(t < kp // 2 - 1)
            def _():
                prefetch(2 * t + 2, 0)
            scat(1)
            return carry

        lax.fori_loop(0, kp // 2, outer, 0)

        if kp < common:
            prefetch(kp, 0)
            wait_l(0)
            if rem > 0:
                @pl.when(wid < rem)
                def _():
                    prefetch(common, 1)
            scat(0)
            if rem > 0:
                @pl.when(wid < rem)
                def _():
                    wait_l(1)
                    scat(1)
        elif rem > 0:
            @pl.when(wid < rem)
            def _():
                prefetch(common, 1)
                wait_l(1)
                scat(1)
        plsc.subcore_barrier()

        pltpu.sync_copy(shared.at[stripe], out_hbm.at[cid, stripe])

    return pl.kernel(
        body,
        out_type=jax.ShapeDtypeStruct((NC, NPAD, D), _f32),
        mesh=_sc_mesh(),
        scratch_types=[
            pltpu.VMEM((CH,), jnp.int32),
            pltpu.VMEM((CH,), jnp.int32),
            pltpu.VMEM((CH, D), _f32),
            pltpu.VMEM((CH, D), _f32),
            pltpu.VMEM_SHARED((NPAD, D), _f32),
            pltpu.SemaphoreType.DMA,
            pltpu.SemaphoreType.DMA,
        ],
    )


def _sc_scatter(e_half, dst, init, ne, ioff, init_per_core):
    return _scatter_kernel(ne, ioff, init_per_core)(e_half, dst, init)


def _dot(a, b):
    return jnp.dot(a, b, preferred_element_type=_f32)


def _proj_body(x_ref, w_ref, p_ref, q_ref):
    pq = _dot(x_ref[...], w_ref[...])
    p_ref[...] = pq[:, :D]
    q_ref[...] = pq[:, D:]


def _proj(x, w_cat, bn=2000):
    grid = (N // bn,)
    return pl.pallas_call(
        _proj_body,
        grid=grid,
        in_specs=[
            pl.BlockSpec((bn, D), lambda i: (i, 0)),
            pl.BlockSpec((D, 2 * D), lambda i: (0, 0)),
        ],
        out_specs=[
            pl.BlockSpec((bn, D), lambda i: (i, 0)),
            pl.BlockSpec((bn, D), lambda i: (i, 0)),
        ],
        out_shape=[
            jax.ShapeDtypeStruct((N, D), _f32),
            jax.ShapeDtypeStruct((N, D), _f32),
        ],
        compiler_params=pltpu.CompilerParams(
            dimension_semantics=("parallel",)),
    )(x, w_cat)


def _mlp_tail(h, w2_ref, b2_ref, w3_ref, b3_ref, g_ref, bt_ref):
    h = jnp.maximum(_dot(h, w2_ref[...]) + b2_ref[...], 0.0)
    h = _dot(h, w3_ref[...]) + b3_ref[...]
    mu = jnp.mean(h, axis=-1, keepdims=True)
    d = h - mu
    var = jnp.mean(d * d, axis=-1, keepdims=True)
    return (d * lax.rsqrt(var + 1e-5)) * g_ref[...] + bt_ref[...]


def _edge_body(e_ref, u_ref, w1_ref, b1_ref, w2_ref, b2_ref,
               w3_ref, b3_ref, g_ref, bt_ref, out_ref):
    h = _dot(e_ref[...], w1_ref[...]) + u_ref[...] + b1_ref[...]
    h = jnp.maximum(h, 0.0)
    out_ref[...] = _mlp_tail(h, w2_ref, b2_ref, w3_ref, b3_ref,
                             g_ref, bt_ref) + e_ref[...]


def _edge(e, u, w1a, b1, w2, b2, w3, b3, g, bt, ne, eoff, be=2000):
    grid = (ne // be,)
    blk = eoff // be
    erow = lambda i: (i + blk, 0)
    row = lambda i: (i, 0)
    full = lambda i: (0, 0)
    return pl.pallas_call(
        _edge_body,
        grid=grid,
        in_specs=[
            pl.BlockSpec((be, D), erow),
            pl.BlockSpec((be, D), row),
            pl.BlockSpec((D, D), full),
            pl.BlockSpec((1, D), full),
            pl.BlockSpec((D, D), full),
            pl.BlockSpec((1, D), full),
            pl.BlockSpec((D, D), full),
            pl.BlockSpec((1, D), full),
            pl.BlockSpec((1, D), full),
            pl.BlockSpec((1, D), full),
        ],
        out_specs=pl.BlockSpec((be, D), row),
        out_shape=jax.ShapeDtypeStruct((ne, D), _f32),
        compiler_params=pltpu.CompilerParams(
            dimension_semantics=("parallel",)),
    )(e, u, w1a, b1, w2, b2, w3, b3, g, bt)


def _node_body(a0_ref, a1_ref, x_ref, wa_ref, wx_ref, b1_ref, w2_ref, b2_ref,
               w3_ref, b3_ref, g_ref, bt_ref, out_ref):
    s = a0_ref[...] + a1_ref[...]
    h = _dot(s, wa_ref[...]) + _dot(x_ref[...], wx_ref[...]) + b1_ref[...]
    h = jnp.maximum(h, 0.0)
    out_ref[...] = _mlp_tail(h, w2_ref, b2_ref, w3_ref, b3_ref,
                             g_ref, bt_ref) + x_ref[...]


def _node_proj_body(a0_ref, a1_ref, x_ref, wa_ref, wx_ref, b1_ref, w2_ref,
                    b2_ref, w3_ref, b3_ref, g_ref, bt_ref, wn_ref,
                    out_ref, p_ref, q_ref):
    s = a0_ref[...] + a1_ref[...]
    h = _dot(s, wa_ref[...]) + _dot(x_ref[...], wx_ref[...]) + b1_ref[...]
    h = jnp.maximum(h, 0.0)
    xn = _mlp_tail(h, w2_ref, b2_ref, w3_ref, b3_ref,
                   g_ref, bt_ref) + x_ref[...]
    out_ref[...] = xn
    pq = _dot(xn, wn_ref[...])
    p_ref[...] = pq[:, :D]
    q_ref[...] = pq[:, D:]


def _node(a0, a1, x, wa, wx, b1, w2, b2, w3, b3, g, bt, wnext=None, bn=2000):
    grid = (N // bn,)
    row = lambda i: (i, 0)
    full = lambda i: (0, 0)
    in_specs = [
        pl.BlockSpec((bn, D), row),
        pl.BlockSpec((bn, D), row),
        pl.BlockSpec((bn, D), row),
        pl.BlockSpec((D, D), full),
        pl.BlockSpec((D, D), full),
        pl.BlockSpec((1, D), full),
        pl.BlockSpec((D, D), full),
        pl.BlockSpec((1, D), full),
        pl.BlockSpec((D, D), full),
        pl.BlockSpec((1, D), full),
        pl.BlockSpec((1, D), full),
        pl.BlockSpec((1, D), full),
    ]
    args = (a0, a1, x, wa, wx, b1, w2, b2, w3, b3, g, bt)
    if wnext is None:
        return pl.pallas_call(
            _node_body,
            grid=grid,
            in_specs=in_specs,
            out_specs=pl.BlockSpec((bn, D), row),
            out_shape=jax.ShapeDtypeStruct((N, D), _f32),
            compiler_params=pltpu.CompilerParams(
                dimension_semantics=("parallel",)),
        )(*args)
    return pl.pallas_call(
        _node_proj_body,
        grid=grid,
        in_specs=in_specs + [pl.BlockSpec((D, 2 * D), full)],
        out_specs=[
            pl.BlockSpec((bn, D), row),
            pl.BlockSpec((bn, D), row),
            pl.BlockSpec((bn, D), row),
        ],
        out_shape=[
            jax.ShapeDtypeStruct((N, D), _f32),
            jax.ShapeDtypeStruct((N, D), _f32),
            jax.ShapeDtypeStruct((N, D), _f32),
        ],
        compiler_params=pltpu.CompilerParams(
            dimension_semantics=("parallel",)),
    )(*args, wnext)


def kernel(node_features, edge_features, edge_index,
           e_w1, e_b1, e_w2, e_b2, e_w3, e_b3, e_g, e_bt,
           n_w1, n_b1, n_w2, n_b2, n_w3, n_b3, n_g, n_bt):
    src = edge_index[0]
    dst = edge_index[1]
    x = node_features
    zeros = jnp.zeros((NPAD, D), _f32)
    num_passes = e_w1.shape[0]
    r = lambda v: v.reshape(1, D)

    def wcat(i):
        w1 = e_w1[i]
        return jnp.concatenate([w1[D:2 * D], w1[2 * D:]], axis=1)

    esplit = [None] * len(SPLITS)
    first = True
    p, q = _proj(x, wcat(0))
    for i in range(num_passes):
        ew = (e_w1[i][:D], r(e_b1[i]), e_w2[i], r(e_b2[i]),
              e_w3[i], r(e_b3[i]), r(e_g[i]), r(e_bt[i]))
        us = [_sc_gather(p, q, src, dst, ne, off)
              for ne, off in zip(SPLITS, SPLIT_OFFS)]
        for j, (ne, off) in enumerate(zip(SPLITS, SPLIT_OFFS)):
            if first:
                esplit[j] = _edge(edge_features, us[j], *ew, ne=ne, eoff=off)
            else:
                esplit[j] = _edge(esplit[j], us[j], *ew, ne=ne, eoff=0)
        first = False
        part = zeros
        for j, (ne, off) in enumerate(zip(SPLITS, SPLIT_OFFS)):
            part = _sc_scatter(esplit[j], dst, part, ne, off, j > 0)
        nw1 = n_w1[i]
        nargs = (part[0], part[1], x, nw1[:D], nw1[D:], r(n_b1[i]),
                 n_w2[i], r(n_b2[i]), n_w3[i], r(n_b3[i]),
                 r(n_g[i]), r(n_bt[i]))
        if i + 1 < num_passes:
            x, p, q = _node(*nargs, wnext=wcat(i + 1))
        else:
            x = _node(*nargs)
    return x
